# baseline (speedup 1.0000x reference)
import functools

import jax
import jax.numpy as jnp
from jax import lax
from jax.experimental import pallas as pl
from jax.experimental.pallas import tpu as pltpu
from jax.experimental.pallas import tpu_sc as plsc

B = 8
N = 20000
FIGSIZE = 640.0
CONF_THRESH = 0.2
NEG = -1e30

NW = 32
ROWS = 624
BUF_ROWS = 656
BUF = BUF_ROWS * 7
GROUPS = BUF_ROWS // 16

_mesh = plsc.VectorSubcoreMesh(core_axis_name="c", subcore_axis_name="s")


@functools.partial(
    pl.kernel,
    mesh=_mesh,
    out_type=jax.ShapeDtypeStruct((NW, 16), jnp.float32),
    compiler_params=pltpu.CompilerParams(needs_layout_passes=False),
    scratch_types=[
        pltpu.VMEM((BUF,), jnp.float32),
        pltpu.VMEM((BUF,), jnp.float32),
        pltpu.VMEM((B * 6 * 16,), jnp.float32),
        pltpu.VMEM((16,), jnp.float32),
        pltpu.SemaphoreType.DMA,
        pltpu.SemaphoreType.DMA,
    ],
)
def _sc_partial_max(boxes_hbm, params_hbm, out_hbm, buf0, buf1, par_v, res_v,
                    sem0, sem1):
    wid = lax.axis_index("c") * 16 + lax.axis_index("s")

    pltpu.sync_copy(params_hbm, par_v)

    lane = lax.iota(jnp.int32, 16)
    idx7 = lane * 7

    def off(b):
        return (b * N + wid * ROWS) * 7

    res = jnp.full((16,), NEG, jnp.float32)
    handles = {0: pltpu.async_copy(boxes_hbm.at[pl.ds(off(0), BUF)], buf0,
                                   sem0)}
    for b in range(B):
        cur = buf0 if b % 2 == 0 else buf1
        if b + 1 < B:
            nbuf = buf1 if b % 2 == 0 else buf0
            nsem = sem1 if b % 2 == 0 else sem0
            handles[b + 1] = pltpu.async_copy(
                boxes_hbm.at[pl.ds(off(b + 1), BUF)], nbuf, nsem)
        handles[b].wait()

        gx1 = par_v[pl.ds((b * 6 + 0) * 16, 16)]
        gy1 = par_v[pl.ds((b * 6 + 1) * 16, 16)]
        gx2 = par_v[pl.ds((b * 6 + 2) * 16, 16)]
        gy2 = par_v[pl.ds((b * 6 + 3) * 16, 16)]
        area2 = par_v[pl.ds((b * 6 + 4) * 16, 16)]
        thr = par_v[pl.ds((b * 6 + 5) * 16, 16)]

        def body(g, macc):
            base = idx7 + g * 112
            cx = plsc.load_gather(cur, [base])
            cy = plsc.load_gather(cur, [base + 1])
            bw = plsc.load_gather(cur, [base + 2])
            bh = plsc.load_gather(cur, [base + 3])
            conf = plsc.load_gather(cur, [base + 4])
            cls_f = plsc.load_gather(cur, [base + 6])
            x1 = (cx - bw / 2.0) * FIGSIZE
            y1 = (cy - bh / 2.0) * FIGSIZE
            x2 = (cx + bw / 2.0) * FIGSIZE
            y2 = (cy + bh / 2.0) * FIGSIZE
            ix1 = jnp.maximum(x1, gx1)
            iy1 = jnp.maximum(y1, gy1)
            ix2 = jnp.minimum(x2, gx2)
            iy2 = jnp.minimum(y2, gy2)
            inter = jnp.maximum(ix2 - ix1, 0.0) * jnp.maximum(iy2 - iy1, 0.0)
            area1 = (x2 - x1) * (y2 - y1)
            ious = inter / (area1 + area2 - inter)
            valid = ((conf > CONF_THRESH) & (ious >= thr)
                     & (cls_f.astype(jnp.int32) == 0))
            return jnp.maximum(macc, jnp.where(valid, conf, NEG))

        mx = jnp.max(lax.fori_loop(0, GROUPS, body, jnp.full((16,), NEG,
                                                             jnp.float32)))
        res = jnp.where(lane == b, mx, res)

    res_v[...] = res
    pltpu.sync_copy(res_v, out_hbm.at[wid])


def kernel(boxes, gt, iou_thresh):
    flat = boxes.reshape(-1)
    gx1, gy1, gx2, gy2 = gt[:, 0], gt[:, 1], gt[:, 2], gt[:, 3]
    area2 = (gx2 - gx1) * (gy2 - gy1)
    thr = jnp.broadcast_to(jnp.asarray(iou_thresh, jnp.float32), (B,))
    params = jnp.stack([gx1, gy1, gx2, gy2, area2, thr], axis=1)
    params = jnp.repeat(params[:, :, None], 16, axis=2).reshape(-1)
    partials = _sc_partial_max(flat, params)
    mx = jnp.max(partials, axis=0)[:B]
    chosen = jnp.where(mx > NEG, mx, 0.0)
    return jnp.mean(chosen), chosen
